# baseline (speedup 1.0000x reference)
import jax
import jax.numpy as jnp
from jax.experimental import pallas as pl

_N = 4194304
_COLS = 1024
_ROWS = _N // _COLS
_BLOCK_ROWS = 512


def _body(o_ref, t_ref, m_ref, r_ref):
    d = o_ref[...] - t_ref[...]
    r_ref[...] = jnp.where(m_ref[...], d * d, 0.0)


def kernel(outputs, targets, precondition):
    o2 = outputs.reshape(_ROWS, _COLS)
    t2 = targets.reshape(_ROWS, _COLS)
    m2 = precondition.reshape(_ROWS, _COLS)
    spec = pl.BlockSpec((_BLOCK_ROWS, _COLS), lambda i: (i, 0))
    out = pl.pallas_call(
        _body,
        grid=(_ROWS // _BLOCK_ROWS,),
        in_specs=[spec, spec, spec],
        out_specs=spec,
        out_shape=jax.ShapeDtypeStruct((_ROWS, _COLS), jnp.float32),
    )(o2, t2, m2)
    return out.reshape(_N, 1)


# speedup vs baseline: 6.0263x; 6.0263x over previous
import jax
import jax.numpy as jnp
from jax.experimental import pallas as pl

_N = 4194304
_BLOCK = 524288


def _body(o_ref, t_ref, m_ref, r_ref):
    d = o_ref[...] - t_ref[...]
    r_ref[...] = jnp.where(m_ref[...], d * d, 0.0)


def kernel(outputs, targets, precondition):
    m = precondition.reshape(_N)
    spec = pl.BlockSpec((_BLOCK,), lambda i: (i,))
    out = pl.pallas_call(
        _body,
        grid=(_N // _BLOCK,),
        in_specs=[spec, spec, spec],
        out_specs=spec,
        out_shape=jax.ShapeDtypeStruct((_N,), jnp.float32),
    )(outputs, targets, m)
    return out.reshape(_N, 1)
